# Initial kernel scaffold; baseline (speedup 1.0000x reference)
import jax
import jax.numpy as jnp
from jax.experimental import pallas as pl


def kernel(x, W):
    raise NotImplementedError("write your pallas kernel here")


# speedup vs baseline: 3.4106x; 3.4106x over previous
import jax
import jax.numpy as jnp
from jax.experimental import pallas as pl


def _copy_block(w_ref, o_ref):
    o_ref[...] = w_ref[...]


def kernel(x, W):
    seq_len = x.shape[1]
    n_model = W.shape[1]
    blk = 512
    out = pl.pallas_call(
        _copy_block,
        grid=(seq_len // blk,),
        in_specs=[pl.BlockSpec((blk, n_model), lambda i: (i, 0))],
        out_specs=pl.BlockSpec((blk, n_model), lambda i: (i, 0)),
        out_shape=jax.ShapeDtypeStruct((seq_len, n_model), W.dtype),
    )(W)
    return out
